# baseline (speedup 1.0000x reference)
import functools

import jax
import jax.numpy as jnp
import numpy as np
from jax import lax
from jax.experimental import pallas as pl
from jax.experimental.pallas import tpu as pltpu
from jax.experimental.pallas import tpu_sc as plsc

_ERROR_PROB = 0.01
_B, _L, _V = 2048, 50, 128
_ROWS = _B * _L
_INV = 1.0 / (_V - 2)

_NC, _NS = 2, 16
_NW = _NC * _NS
_BPW = _B // _NW


def _threefry2x32_np(k1, k2, x0, x1):
    def rotl(x, r):
        return ((x << np.uint32(r)) | (x >> np.uint32(32 - r))).astype(np.uint32)

    ks = [np.uint32(k1), np.uint32(k2),
          np.uint32(np.uint32(k1) ^ np.uint32(k2) ^ np.uint32(0x1BD11BDA))]
    x0 = (x0 + ks[0]).astype(np.uint32)
    x1 = (x1 + ks[1]).astype(np.uint32)
    rots = [(13, 15, 26, 6), (17, 29, 16, 24)]
    for i in range(5):
        for r in rots[i % 2]:
            x0 = (x0 + x1).astype(np.uint32)
            x1 = rotl(x1, r)
            x1 = (x1 ^ x0).astype(np.uint32)
        x0 = (x0 + ks[(i + 1) % 3]).astype(np.uint32)
        x1 = (x1 + ks[(i + 2) % 3] + np.uint32(i + 1)).astype(np.uint32)
    return x0, x1


@functools.cache
def _row_mask_f32() -> np.ndarray:
    n = _ROWS * (_V - 1)
    lo = np.arange(n, dtype=np.uint32)
    hi = np.zeros(n, np.uint32)
    a, b = _threefry2x32_np(np.uint32(0), np.uint32(42), hi, lo)
    bits = a ^ b
    fbits = ((bits >> np.uint32(9)) | np.uint32(0x3F800000)).view(np.float32)
    u = fbits - np.float32(1.0)
    mask = np.any(u.reshape(_ROWS, _V - 1) < np.float32(_ERROR_PROB), axis=1)
    return np.ascontiguousarray(mask.astype(np.float32).reshape(_B, _L, 1))


_CB = 128
_NCH = _B // _CB
_D = 3


_Q = 4
_QB = _CB // _Q


def _ring_body(mask_hbm, m_hbm, p_hbm, mo_hbm, po_hbm,
               mbuf, pbuf, kbuf, mobuf, pobuf, rsem, wsem):
    def _sub(hbm, buf, c, s, sem, row, op):
        for q in range(_Q):
            sl = pl.ds(c * _CB + q * _QB, _QB)
            bl = pl.ds(q * _QB, _QB)
            cp = pltpu.make_async_copy(hbm.at[sl], buf.at[s].at[bl],
                                       sem.at[row * _Q + q, s])
            getattr(cp, op)()

    def _subw(buf, hbm, c, s, sem, row, op):
        for q in range(_Q):
            sl = pl.ds(c * _CB + q * _QB, _QB)
            bl = pl.ds(q * _QB, _QB)
            cp = pltpu.make_async_copy(buf.at[s].at[bl], hbm.at[sl],
                                       sem.at[row * _Q + q, s])
            getattr(cp, op)()

    def rd(c, s):
        _sub(m_hbm, mbuf, c, s, rsem, 0, "start")
        _sub(p_hbm, pbuf, c, s, rsem, 1, "start")
        pltpu.make_async_copy(mask_hbm.at[pl.ds(c * _CB, _CB)], kbuf.at[s],
                              rsem.at[2 * _Q, s]).start()

    def rd_wait(c, s):
        _sub(m_hbm, mbuf, c, s, rsem, 0, "wait")
        _sub(p_hbm, pbuf, c, s, rsem, 1, "wait")
        pltpu.make_async_copy(mask_hbm.at[pl.ds(c * _CB, _CB)], kbuf.at[s],
                              rsem.at[2 * _Q, s]).wait()

    def wr(c, s):
        _subw(mobuf, mo_hbm, c, s, wsem, 0, "start")
        _subw(pobuf, po_hbm, c, s, wsem, 1, "start")

    def wr_wait(c, s):
        _subw(mobuf, mo_hbm, c, s, wsem, 0, "wait")
        _subw(pobuf, po_hbm, c, s, wsem, 1, "wait")

    for cc in range(_D):
        rd(cc, cc)

    def step(c, carry):
        s = lax.rem(c, _D)
        rd_wait(c, s)

        @pl.when(c >= _D)
        def _():
            wr_wait(c - _D, s)

        m = mbuf.at[s][...]
        p = pbuf.at[s][...]
        mask = kbuf.at[s][...]
        m0 = m[:, :, :1]
        p0 = p[:, :, :1]
        repl = (1.0 - m - m0) * _INV
        m_new = jnp.where(mask > 0.5, repl, m)
        p_new = p * (1.0 - _ERROR_PROB) + (1.0 - p - p0) * (_ERROR_PROB * _INV)
        col = jax.lax.broadcasted_iota(jnp.int32, m.shape, 2)
        is0 = col == 0
        mobuf.at[s][...] = jnp.where(is0, m, m_new)
        pobuf.at[s][...] = jnp.where(is0, p, p_new)
        wr(c, s)

        @pl.when(c + _D < _NCH)
        def _():
            rd(c + _D, s)

        return carry

    lax.fori_loop(0, _NCH, step, 0)
    for ss in range(_D):
        wr_wait(_NCH - _D + ss, lax.rem(jnp.int32(_NCH - _D + ss), _D))


def _ring_transform(mask, messages, probs):
    out = jax.ShapeDtypeStruct((_B, _L, _V), jnp.float32)
    anyspec = pl.BlockSpec(memory_space=pl.ANY)
    return pl.pallas_call(
        _ring_body,
        in_specs=[anyspec, anyspec, anyspec],
        out_specs=[anyspec, anyspec],
        out_shape=[out, out],
        scratch_shapes=[
            pltpu.VMEM((_D, _CB, _L, _V), jnp.float32),
            pltpu.VMEM((_D, _CB, _L, _V), jnp.float32),
            pltpu.VMEM((_D, _CB, _L, 1), jnp.float32),
            pltpu.VMEM((_D, _CB, _L, _V), jnp.float32),
            pltpu.VMEM((_D, _CB, _L, _V), jnp.float32),
            pltpu.SemaphoreType.DMA((2 * _Q + 1, _D)),
            pltpu.SemaphoreType.DMA((2 * _Q, _D)),
        ],
    )(mask, messages, probs)


def _m_body(mask_ref, m_ref, mo_ref, mc_ref):
    m = m_ref[...]
    mask = mask_ref[...]
    m0 = m[:, :, :1]
    repl = (1.0 - m - m0) * _INV
    m_new = jnp.where(mask > 0.5, repl, m)
    col = jax.lax.broadcasted_iota(jnp.int32, m.shape, 2)
    is0 = col == 0
    mo_ref[...] = jnp.where(is0, m, m_new)
    mc_ref[...] = m


def _p_body(p_ref, po_ref, pc_ref):
    p = p_ref[...]
    p0 = p[:, :, :1]
    p_new = p * (1.0 - _ERROR_PROB) + (1.0 - p - p0) * (_ERROR_PROB * _INV)
    col = jax.lax.broadcasted_iota(jnp.int32, p.shape, 2)
    is0 = col == 0
    po_ref[...] = jnp.where(is0, p, p_new)
    pc_ref[...] = p


def _tc_transform(mask, messages, probs):
    b, l, v = messages.shape
    bblk = 128
    grid = b // bblk
    blk = pl.BlockSpec((bblk, l, v), lambda i: (i, 0, 0))
    out = jax.ShapeDtypeStruct((b, l, v), jnp.float32)
    m1, mc = pl.pallas_call(
        _m_body,
        grid=(grid,),
        in_specs=[pl.BlockSpec((bblk, l, 1), lambda i: (i, 0, 0)), blk],
        out_specs=[blk, blk],
        out_shape=[out, out],
    )(mask, messages)
    p1, pc = pl.pallas_call(
        _p_body,
        grid=(grid,),
        in_specs=[blk],
        out_specs=[blk, blk],
        out_shape=[out, out],
    )(probs)
    return m1, mc, p1, pc


def _make_sc_copy_kernel():
    mesh = plsc.VectorSubcoreMesh(core_axis_name="c", subcore_axis_name="s")
    out = jax.ShapeDtypeStruct((_B, _L, _V), jnp.float32)

    @functools.partial(
        pl.kernel,
        out_type=[out, out],
        mesh=mesh,
    )
    def k(m_hbm, p_hbm, mc_hbm, pc_hbm):
        wid = lax.axis_index("s") * _NC + lax.axis_index("c")
        b0 = wid * _BPW
        sl = pl.ds(b0, _BPW)
        pltpu.sync_copy(m_hbm.at[sl], mc_hbm.at[sl])
        pltpu.sync_copy(p_hbm.at[sl], pc_hbm.at[sl])

    return k


def kernel(messages, probs):
    mask = jnp.asarray(_row_mask_f32())
    m1, p1 = _ring_transform(mask, messages, probs)
    return (m1, p1, messages, probs)


# speedup vs baseline: 1.0051x; 1.0051x over previous
import functools

import jax
import jax.numpy as jnp
import numpy as np
from jax import lax
from jax.experimental import pallas as pl
from jax.experimental.pallas import tpu as pltpu

_ERROR_PROB = 0.01
_B, _L, _V = 2048, 50, 128
_ROWS = _B * _L
_INV = 1.0 / (_V - 2)

_CB = 64
_NCH = _B // _CB
_D = 4


def _threefry2x32_np(k1, k2, x0, x1):
    def rotl(x, r):
        return ((x << np.uint32(r)) | (x >> np.uint32(32 - r))).astype(np.uint32)

    ks = [np.uint32(k1), np.uint32(k2),
          np.uint32(np.uint32(k1) ^ np.uint32(k2) ^ np.uint32(0x1BD11BDA))]
    x0 = (x0 + ks[0]).astype(np.uint32)
    x1 = (x1 + ks[1]).astype(np.uint32)
    rots = [(13, 15, 26, 6), (17, 29, 16, 24)]
    for i in range(5):
        for r in rots[i % 2]:
            x0 = (x0 + x1).astype(np.uint32)
            x1 = rotl(x1, r)
            x1 = (x1 ^ x0).astype(np.uint32)
        x0 = (x0 + ks[(i + 1) % 3]).astype(np.uint32)
        x1 = (x1 + ks[(i + 2) % 3] + np.uint32(i + 1)).astype(np.uint32)
    return x0, x1


@functools.cache
def _row_mask_f32() -> np.ndarray:
    n = _ROWS * (_V - 1)
    lo = np.arange(n, dtype=np.uint32)
    hi = np.zeros(n, np.uint32)
    a, b = _threefry2x32_np(np.uint32(0), np.uint32(42), hi, lo)
    bits = a ^ b
    fbits = ((bits >> np.uint32(9)) | np.uint32(0x3F800000)).view(np.float32)
    u = fbits - np.float32(1.0)
    mask = np.any(u.reshape(_ROWS, _V - 1) < np.float32(_ERROR_PROB), axis=1)
    return np.ascontiguousarray(mask.astype(np.float32).reshape(_B, _L, 1))


def _ring_body(mask_hbm, m_hbm, p_hbm, mo_hbm, po_hbm,
               mbuf, pbuf, kbuf, mobuf, pobuf, rsem, wsem):
    def rd(c, s):
        sl = pl.ds(c * _CB, _CB)
        pltpu.make_async_copy(m_hbm.at[sl], mbuf.at[s], rsem.at[0, s]).start()
        pltpu.make_async_copy(p_hbm.at[sl], pbuf.at[s], rsem.at[1, s]).start()
        pltpu.make_async_copy(mask_hbm.at[sl], kbuf.at[s], rsem.at[2, s]).start()

    def rd_wait(c, s):
        sl = pl.ds(c * _CB, _CB)
        pltpu.make_async_copy(m_hbm.at[sl], mbuf.at[s], rsem.at[0, s]).wait()
        pltpu.make_async_copy(p_hbm.at[sl], pbuf.at[s], rsem.at[1, s]).wait()
        pltpu.make_async_copy(mask_hbm.at[sl], kbuf.at[s], rsem.at[2, s]).wait()

    def wr(c, s):
        sl = pl.ds(c * _CB, _CB)
        pltpu.make_async_copy(mobuf.at[s], mo_hbm.at[sl], wsem.at[0, s]).start()
        pltpu.make_async_copy(pobuf.at[s], po_hbm.at[sl], wsem.at[1, s]).start()

    def wr_wait(c, s):
        sl = pl.ds(c * _CB, _CB)
        pltpu.make_async_copy(mobuf.at[s], mo_hbm.at[sl], wsem.at[0, s]).wait()
        pltpu.make_async_copy(pobuf.at[s], po_hbm.at[sl], wsem.at[1, s]).wait()

    for cc in range(_D):
        rd(cc, cc)

    def step(c, carry):
        s = lax.rem(c, _D)
        rd_wait(c, s)

        @pl.when(c >= _D)
        def _():
            wr_wait(c - _D, s)

        m = mbuf.at[s][...]
        p = pbuf.at[s][...]
        mask = kbuf.at[s][...]
        m0 = m[:, :, :1]
        p0 = p[:, :, :1]
        repl = (1.0 - m - m0) * _INV
        m_new = jnp.where(mask > 0.5, repl, m)
        p_new = p * (1.0 - _ERROR_PROB) + (1.0 - p - p0) * (_ERROR_PROB * _INV)
        col = jax.lax.broadcasted_iota(jnp.int32, m.shape, 2)
        is0 = col == 0
        mobuf.at[s][...] = jnp.where(is0, m, m_new)
        pobuf.at[s][...] = jnp.where(is0, p, p_new)
        wr(c, s)

        @pl.when(c + _D < _NCH)
        def _():
            rd(c + _D, s)

        return carry

    lax.fori_loop(0, _NCH, step, 0)
    for ss in range(_D):
        wr_wait(_NCH - _D + ss, lax.rem(jnp.int32(_NCH - _D + ss), _D))


def _ring_transform(mask, messages, probs):
    out = jax.ShapeDtypeStruct((_B, _L, _V), jnp.float32)
    anyspec = pl.BlockSpec(memory_space=pl.ANY)
    return pl.pallas_call(
        _ring_body,
        in_specs=[anyspec, anyspec, anyspec],
        out_specs=[anyspec, anyspec],
        out_shape=[out, out],
        scratch_shapes=[
            pltpu.VMEM((_D, _CB, _L, _V), jnp.float32),
            pltpu.VMEM((_D, _CB, _L, _V), jnp.float32),
            pltpu.VMEM((_D, _CB, _L, 1), jnp.float32),
            pltpu.VMEM((_D, _CB, _L, _V), jnp.float32),
            pltpu.VMEM((_D, _CB, _L, _V), jnp.float32),
            pltpu.SemaphoreType.DMA((3, _D)),
            pltpu.SemaphoreType.DMA((2, _D)),
        ],
    )(mask, messages, probs)


def kernel(messages, probs):
    mask = jnp.asarray(_row_mask_f32())
    m1, p1 = _ring_transform(mask, messages, probs)
    return (m1, p1, messages, probs)


# speedup vs baseline: 1.0054x; 1.0004x over previous
import functools

import jax
import jax.numpy as jnp
import numpy as np
from jax import lax
from jax.experimental import pallas as pl
from jax.experimental.pallas import tpu as pltpu

_ERROR_PROB = 0.01
_B, _L, _V = 2048, 50, 128
_ROWS = _B * _L
_INV = 1.0 / (_V - 2)

_CB = 64
_NCH = _B // _CB
_D = 6


def _threefry2x32_np(k1, k2, x0, x1):
    def rotl(x, r):
        return ((x << np.uint32(r)) | (x >> np.uint32(32 - r))).astype(np.uint32)

    ks = [np.uint32(k1), np.uint32(k2),
          np.uint32(np.uint32(k1) ^ np.uint32(k2) ^ np.uint32(0x1BD11BDA))]
    x0 = (x0 + ks[0]).astype(np.uint32)
    x1 = (x1 + ks[1]).astype(np.uint32)
    rots = [(13, 15, 26, 6), (17, 29, 16, 24)]
    for i in range(5):
        for r in rots[i % 2]:
            x0 = (x0 + x1).astype(np.uint32)
            x1 = rotl(x1, r)
            x1 = (x1 ^ x0).astype(np.uint32)
        x0 = (x0 + ks[(i + 1) % 3]).astype(np.uint32)
        x1 = (x1 + ks[(i + 2) % 3] + np.uint32(i + 1)).astype(np.uint32)
    return x0, x1


@functools.cache
def _row_mask_f32() -> np.ndarray:
    n = _ROWS * (_V - 1)
    lo = np.arange(n, dtype=np.uint32)
    hi = np.zeros(n, np.uint32)
    a, b = _threefry2x32_np(np.uint32(0), np.uint32(42), hi, lo)
    bits = a ^ b
    fbits = ((bits >> np.uint32(9)) | np.uint32(0x3F800000)).view(np.float32)
    u = fbits - np.float32(1.0)
    mask = np.any(u.reshape(_ROWS, _V - 1) < np.float32(_ERROR_PROB), axis=1)
    return np.ascontiguousarray(mask.astype(np.float32).reshape(_B, _L, 1))


def _ring_body(mask_hbm, m_hbm, p_hbm, mo_hbm, po_hbm,
               mbuf, pbuf, kbuf, mobuf, pobuf, rsem, wsem):
    def rd(c, s):
        sl = pl.ds(c * _CB, _CB)
        pltpu.make_async_copy(m_hbm.at[sl], mbuf.at[s], rsem.at[0, s]).start()
        pltpu.make_async_copy(p_hbm.at[sl], pbuf.at[s], rsem.at[1, s]).start()
        pltpu.make_async_copy(mask_hbm.at[sl], kbuf.at[s], rsem.at[2, s]).start()

    def rd_wait(c, s):
        sl = pl.ds(c * _CB, _CB)
        pltpu.make_async_copy(m_hbm.at[sl], mbuf.at[s], rsem.at[0, s]).wait()
        pltpu.make_async_copy(p_hbm.at[sl], pbuf.at[s], rsem.at[1, s]).wait()
        pltpu.make_async_copy(mask_hbm.at[sl], kbuf.at[s], rsem.at[2, s]).wait()

    def wr(c, s):
        sl = pl.ds(c * _CB, _CB)
        pltpu.make_async_copy(mobuf.at[s], mo_hbm.at[sl], wsem.at[0, s]).start()
        pltpu.make_async_copy(pobuf.at[s], po_hbm.at[sl], wsem.at[1, s]).start()

    def wr_wait(c, s):
        sl = pl.ds(c * _CB, _CB)
        pltpu.make_async_copy(mobuf.at[s], mo_hbm.at[sl], wsem.at[0, s]).wait()
        pltpu.make_async_copy(pobuf.at[s], po_hbm.at[sl], wsem.at[1, s]).wait()

    for cc in range(_D):
        rd(cc, cc)

    def step(c, carry):
        s = lax.rem(c, _D)
        rd_wait(c, s)

        @pl.when(c >= _D)
        def _():
            wr_wait(c - _D, s)

        m = mbuf.at[s][...]
        p = pbuf.at[s][...]
        mask = kbuf.at[s][...]
        m0 = m[:, :, :1]
        p0 = p[:, :, :1]
        repl = (1.0 - m - m0) * _INV
        m_new = jnp.where(mask > 0.5, repl, m)
        p_new = p * (1.0 - _ERROR_PROB) + (1.0 - p - p0) * (_ERROR_PROB * _INV)
        col = jax.lax.broadcasted_iota(jnp.int32, m.shape, 2)
        is0 = col == 0
        mobuf.at[s][...] = jnp.where(is0, m, m_new)
        pobuf.at[s][...] = jnp.where(is0, p, p_new)
        wr(c, s)

        @pl.when(c + _D < _NCH)
        def _():
            rd(c + _D, s)

        return carry

    lax.fori_loop(0, _NCH, step, 0)
    for ss in range(_D):
        wr_wait(_NCH - _D + ss, lax.rem(jnp.int32(_NCH - _D + ss), _D))


def _ring_transform(mask, messages, probs):
    out = jax.ShapeDtypeStruct((_B, _L, _V), jnp.float32)
    anyspec = pl.BlockSpec(memory_space=pl.ANY)
    return pl.pallas_call(
        _ring_body,
        in_specs=[anyspec, anyspec, anyspec],
        out_specs=[anyspec, anyspec],
        out_shape=[out, out],
        scratch_shapes=[
            pltpu.VMEM((_D, _CB, _L, _V), jnp.float32),
            pltpu.VMEM((_D, _CB, _L, _V), jnp.float32),
            pltpu.VMEM((_D, _CB, _L, 1), jnp.float32),
            pltpu.VMEM((_D, _CB, _L, _V), jnp.float32),
            pltpu.VMEM((_D, _CB, _L, _V), jnp.float32),
            pltpu.SemaphoreType.DMA((3, _D)),
            pltpu.SemaphoreType.DMA((2, _D)),
        ],
    )(mask, messages, probs)


def kernel(messages, probs):
    mask = jnp.asarray(_row_mask_f32())
    m1, p1 = _ring_transform(mask, messages, probs)
    return (m1, p1, messages, probs)


# speedup vs baseline: 1.0538x; 1.0481x over previous
import functools

import jax
import jax.numpy as jnp
import numpy as np
from jax import lax
from jax.experimental import pallas as pl
from jax.experimental.pallas import tpu as pltpu

_ERROR_PROB = 0.01
_B, _L, _V = 2048, 50, 128
_ROWS = _B * _L
_INV = 1.0 / (_V - 2)

_CB = 64
_NCH = _B // _CB
_D = 6


def _threefry2x32_np(k1, k2, x0, x1):
    def rotl(x, r):
        return ((x << np.uint32(r)) | (x >> np.uint32(32 - r))).astype(np.uint32)

    ks = [np.uint32(k1), np.uint32(k2),
          np.uint32(np.uint32(k1) ^ np.uint32(k2) ^ np.uint32(0x1BD11BDA))]
    x0 = (x0 + ks[0]).astype(np.uint32)
    x1 = (x1 + ks[1]).astype(np.uint32)
    rots = [(13, 15, 26, 6), (17, 29, 16, 24)]
    for i in range(5):
        for r in rots[i % 2]:
            x0 = (x0 + x1).astype(np.uint32)
            x1 = rotl(x1, r)
            x1 = (x1 ^ x0).astype(np.uint32)
        x0 = (x0 + ks[(i + 1) % 3]).astype(np.uint32)
        x1 = (x1 + ks[(i + 2) % 3] + np.uint32(i + 1)).astype(np.uint32)
    return x0, x1


@functools.cache
def _row_mask_f32() -> np.ndarray:
    n = _ROWS * (_V - 1)
    lo = np.arange(n, dtype=np.uint32)
    hi = np.zeros(n, np.uint32)
    a, b = _threefry2x32_np(np.uint32(0), np.uint32(42), hi, lo)
    bits = a ^ b
    fbits = ((bits >> np.uint32(9)) | np.uint32(0x3F800000)).view(np.float32)
    u = fbits - np.float32(1.0)
    mask = np.any(u.reshape(_ROWS, _V - 1) < np.float32(_ERROR_PROB), axis=1)
    return np.ascontiguousarray(mask.astype(np.float32).reshape(_B, _L))


def _ring_body(mask_hbm, m_hbm, p_hbm, mo_hbm, po_hbm,
               mbuf, pbuf, kbuf, mobuf, pobuf, rsem, wsem):
    def rd(c, s):
        sl = pl.ds(c * _CB, _CB)
        pltpu.make_async_copy(m_hbm.at[sl], mbuf.at[s], rsem.at[0, s]).start()
        pltpu.make_async_copy(p_hbm.at[sl], pbuf.at[s], rsem.at[1, s]).start()
        pltpu.make_async_copy(mask_hbm.at[sl], kbuf.at[s], rsem.at[2, s]).start()

    def rd_wait(c, s):
        sl = pl.ds(c * _CB, _CB)
        pltpu.make_async_copy(m_hbm.at[sl], mbuf.at[s], rsem.at[0, s]).wait()
        pltpu.make_async_copy(p_hbm.at[sl], pbuf.at[s], rsem.at[1, s]).wait()
        pltpu.make_async_copy(mask_hbm.at[sl], kbuf.at[s], rsem.at[2, s]).wait()

    def wr(c, s):
        sl = pl.ds(c * _CB, _CB)
        pltpu.make_async_copy(mobuf.at[s], mo_hbm.at[sl], wsem.at[0, s]).start()
        pltpu.make_async_copy(pobuf.at[s], po_hbm.at[sl], wsem.at[1, s]).start()

    def wr_wait(c, s):
        sl = pl.ds(c * _CB, _CB)
        pltpu.make_async_copy(mobuf.at[s], mo_hbm.at[sl], wsem.at[0, s]).wait()
        pltpu.make_async_copy(pobuf.at[s], po_hbm.at[sl], wsem.at[1, s]).wait()

    for cc in range(_D):
        rd(cc, cc)

    def step(c, carry):
        s = lax.rem(c, _D)
        rd_wait(c, s)

        @pl.when(c >= _D)
        def _():
            wr_wait(c - _D, s)

        m = mbuf.at[s][...]
        p = pbuf.at[s][...]
        mask = kbuf.at[s][...][:, :, None]
        m0 = m[:, :, :1]
        p0 = p[:, :, :1]
        repl = (1.0 - m - m0) * _INV
        m_new = jnp.where(mask > 0.5, repl, m)
        p_new = p * (1.0 - _ERROR_PROB) + (1.0 - p - p0) * (_ERROR_PROB * _INV)
        col = jax.lax.broadcasted_iota(jnp.int32, m.shape, 2)
        is0 = col == 0
        mobuf.at[s][...] = jnp.where(is0, m, m_new)
        pobuf.at[s][...] = jnp.where(is0, p, p_new)
        wr(c, s)

        @pl.when(c + _D < _NCH)
        def _():
            rd(c + _D, s)

        return carry

    lax.fori_loop(0, _NCH, step, 0)
    for ss in range(_D):
        wr_wait(_NCH - _D + ss, lax.rem(jnp.int32(_NCH - _D + ss), _D))


def _ring_transform(mask, messages, probs):
    out = jax.ShapeDtypeStruct((_B, _L, _V), jnp.float32)
    anyspec = pl.BlockSpec(memory_space=pl.ANY)
    return pl.pallas_call(
        _ring_body,
        in_specs=[anyspec, anyspec, anyspec],
        out_specs=[anyspec, anyspec],
        out_shape=[out, out],
        scratch_shapes=[
            pltpu.VMEM((_D, _CB, _L, _V), jnp.float32),
            pltpu.VMEM((_D, _CB, _L, _V), jnp.float32),
            pltpu.VMEM((_D, _CB, _L), jnp.float32),
            pltpu.VMEM((_D, _CB, _L, _V), jnp.float32),
            pltpu.VMEM((_D, _CB, _L, _V), jnp.float32),
            pltpu.SemaphoreType.DMA((3, _D)),
            pltpu.SemaphoreType.DMA((2, _D)),
        ],
    )(mask, messages, probs)


def kernel(messages, probs):
    mask = jnp.asarray(_row_mask_f32())
    m1, p1 = _ring_transform(mask, messages, probs)
    return (m1, p1, messages, probs)
